# baseline (speedup 1.0000x reference)
import functools

import jax
import jax.numpy as jnp
from jax import lax
from jax.experimental import pallas as pl
from jax.experimental.pallas import tpu as pltpu
from jax.experimental.pallas import tpu_sc as plsc

_B = 16384
_D = 16

_info = plsc.get_sparse_core_info()
_NC = _info.num_cores
_NS = _info.num_subcores
_NW = _NC * _NS
_BPW = _B // _NW

_mesh = plsc.VectorSubcoreMesh(core_axis_name="c", subcore_axis_name="s")


@functools.partial(
    pl.kernel,
    mesh=_mesh,
    compiler_params=pltpu.CompilerParams(use_tc_tiling_on_sc=False),
    out_type=(
        jax.ShapeDtypeStruct((_B, _D), jnp.float32),
        jax.ShapeDtypeStruct((_B, _D), jnp.float32),
    ),
    scratch_types=[
        pltpu.VMEM((_BPW,), jnp.int32),
        pltpu.VMEM((_BPW,), jnp.int32),
        pltpu.VMEM((_BPW, _D), jnp.float32),
        pltpu.VMEM((_BPW, _D), jnp.float32),
        pltpu.SemaphoreType.DMA,
        pltpu.SemaphoreType.DMA,
    ],
)
def _dual_gather(ctab, rtab, cids, rids, xr, xc,
                 cidx_v, ridx_v, crows_v, rrows_v, sem_c, sem_r):
    wid = lax.axis_index("s") * _NC + lax.axis_index("c")
    base = wid * _BPW
    pltpu.sync_copy(rids.at[pl.ds(base, _BPW)], ridx_v)
    pltpu.sync_copy(cids.at[pl.ds(base, _BPW)], cidx_v)
    cp_r = pltpu.async_copy(rtab.at[ridx_v], rrows_v, sem_r)
    cp_c = pltpu.async_copy(ctab.at[cidx_v], crows_v, sem_c)
    cp_r.wait()
    pltpu.sync_copy(rrows_v, xr.at[pl.ds(base, _BPW)])
    cp_c.wait()
    pltpu.sync_copy(crows_v, xc.at[pl.ds(base, _BPW)])


def kernel(contributor_table, recipient_table, contributor_ids, recipient_ids):
    xr, xc = _dual_gather(
        contributor_table,
        recipient_table,
        contributor_ids.astype(jnp.int32),
        recipient_ids.astype(jnp.int32),
    )
    return (xr, xc)
